# baseline (speedup 1.0000x reference)
import functools

import jax
import jax.numpy as jnp
from jax import lax
from jax.experimental import pallas as pl
from jax.experimental.pallas import tpu as pltpu
from jax.experimental.pallas import tpu_sc as plsc

B, L = 16384, 200
PAD = -1.0

NC, NS, LANES = 2, 16, 16
NW = NC * NS
ROWS_PER_W = B // NW
CHUNK_ROWS = 128
NCHUNK = ROWS_PER_W // CHUNK_ROWS
GROUPS = CHUNK_ROWS // LANES
SEARCH_ITERS = 8

SLICE_OFFS = tuple(range(0, L - 16, 16)) + (L - 16,)

_mesh = plsc.VectorSubcoreMesh(core_axis_name="c", subcore_axis_name="s")


@functools.partial(
    pl.kernel,
    out_type=[
        jax.ShapeDtypeStruct((B, L), jnp.float32),
        jax.ShapeDtypeStruct((B,), jnp.int32),
    ],
    mesh=_mesh,
    scratch_types=[
        pltpu.VMEM((CHUNK_ROWS, L), jnp.float32),
        pltpu.VMEM((ROWS_PER_W,), jnp.int32),
    ],
    compiler_params=pltpu.CompilerParams(
        needs_layout_passes=False, use_tc_tiling_on_sc=True
    ),
)
def _dense_to_ragged(x_hbm, vals_hbm, len_hbm, buf, len_v):
    wid = lax.axis_index("s") * NC + lax.axis_index("c")
    row_base = wid * ROWS_PER_W
    iota16 = lax.iota(jnp.int32, 16)

    for chunk in range(NCHUNK):
        r0 = row_base + chunk * CHUNK_ROWS
        pltpu.sync_copy(x_hbm.at[pl.ds(r0, CHUNK_ROWS)], buf)

        def grp(g, _):
            rows = g * LANES + iota16
            lo = jnp.zeros((16,), jnp.int32)
            hi = jnp.full((16,), L, jnp.int32)
            for _i in range(SEARCH_ITERS):
                active = lo < hi
                mid = jnp.right_shift(lo + hi, 1)
                midc = jnp.minimum(mid, L - 1)
                v = plsc.load_gather(buf, [rows, midc])
                is_pad = v == PAD
                lo2 = jnp.where(is_pad, lo, mid + 1)
                hi2 = jnp.where(is_pad, mid, hi)
                lo = jnp.where(active, lo2, lo)
                hi = jnp.where(active, hi2, hi)
            len_v[pl.ds(chunk * CHUNK_ROWS + g * LANES, 16)] = lo
            return 0

        lax.fori_loop(0, GROUPS, grp, 0)

        def ew(r, _):
            for o in SLICE_OFFS:
                v = buf[r, pl.ds(o, 16)]
                buf[r, pl.ds(o, 16)] = jnp.where(v == PAD, jnp.float32(0.0), v)
            return 0

        lax.fori_loop(0, CHUNK_ROWS, ew, 0)

        pltpu.sync_copy(buf, vals_hbm.at[pl.ds(r0, CHUNK_ROWS)])

    pltpu.sync_copy(len_v, len_hbm.at[pl.ds(wid * ROWS_PER_W, ROWS_PER_W)])


def kernel(inputs):
    values, row_lengths = _dense_to_ragged(inputs)
    return values, row_lengths


# speedup vs baseline: 1.3497x; 1.3497x over previous
import functools

import jax
import jax.numpy as jnp
from jax import lax
from jax.experimental import pallas as pl
from jax.experimental.pallas import tpu as pltpu
from jax.experimental.pallas import tpu_sc as plsc

B, L = 16384, 200
PAD = -1.0

NC, NS, LANES = 2, 16, 16
NW = NC * NS
COLS_PER_W = B // NW
CHUNK = 128
NCHUNK = COLS_PER_W // CHUNK
GROUPS_PER_CHUNK = CHUNK // LANES
SEARCH_ITERS = 8

_mesh = plsc.VectorSubcoreMesh(core_axis_name="c", subcore_axis_name="s")


@functools.partial(
    pl.kernel,
    out_type=[
        jax.ShapeDtypeStruct((L, B), jnp.float32),
        jax.ShapeDtypeStruct((B,), jnp.int32),
    ],
    mesh=_mesh,
    scratch_types=[
        pltpu.VMEM((L, COLS_PER_W), jnp.float32),
        pltpu.VMEM((COLS_PER_W,), jnp.int32),
        [pltpu.SemaphoreType.DMA] * NCHUNK,
    ],
    compiler_params=pltpu.CompilerParams(
        needs_layout_passes=False, use_tc_tiling_on_sc=True
    ),
)
def _dense_to_ragged(xt_hbm, vt_hbm, len_hbm, buf, len_v, sems):
    wid = lax.axis_index("s") * NC + lax.axis_index("c")
    b0 = wid * COLS_PER_W
    iota16 = lax.iota(jnp.int32, 16)
    zeros16 = jnp.zeros((16,), jnp.float32)

    ins = [
        pltpu.async_copy(
            xt_hbm.at[:, pl.ds(b0 + i * CHUNK, CHUNK)],
            buf.at[:, pl.ds(i * CHUNK, CHUNK)],
            sems[i],
        )
        for i in range(NCHUNK)
    ]

    outs = []
    for i in range(NCHUNK):
        ins[i].wait()

        def grp(g, _):
            cols = g * LANES + iota16
            lo = jnp.zeros((16,), jnp.int32)
            hi = jnp.full((16,), L, jnp.int32)
            for _i in range(SEARCH_ITERS):
                active = lo < hi
                mid = jnp.right_shift(lo + hi, 1)
                midc = jnp.minimum(mid, L - 1)
                v = plsc.load_gather(buf, [midc, cols])
                is_pad = v == PAD
                lo2 = jnp.where(is_pad, lo, mid + 1)
                hi2 = jnp.where(is_pad, mid, hi)
                lo = jnp.where(active, lo2, lo)
                hi = jnp.where(active, hi2, hi)
            len_v[pl.ds(g * LANES, 16)] = lo

            def zero_tail(l, _):
                lvec = jnp.full((16,), l, dtype=jnp.int32)
                plsc.store_scatter(
                    buf, [lvec, cols], zeros16, mask=lvec >= lo
                )
                return 0

            lax.fori_loop(jnp.min(lo), L, zero_tail, 0)
            return 0

        lax.fori_loop(i * GROUPS_PER_CHUNK, (i + 1) * GROUPS_PER_CHUNK, grp, 0)

        outs.append(
            pltpu.async_copy(
                buf.at[:, pl.ds(i * CHUNK, CHUNK)],
                vt_hbm.at[:, pl.ds(b0 + i * CHUNK, CHUNK)],
                sems[i],
            )
        )

    for h in outs:
        h.wait()
    pltpu.sync_copy(len_v, len_hbm.at[pl.ds(b0, COLS_PER_W)])


def kernel(inputs):
    values_t, row_lengths = _dense_to_ragged(inputs.T)
    return values_t.T, row_lengths


# speedup vs baseline: 2.1006x; 1.5564x over previous
import functools

import jax
import jax.numpy as jnp
from jax import lax
from jax.experimental import pallas as pl
from jax.experimental.pallas import tpu as pltpu
from jax.experimental.pallas import tpu_sc as plsc

B, L = 16384, 200
PAD = -1.0

NC, NS, LANES = 2, 16, 16
NW = NC * NS
COLS_PER_W = B // NW
CHUNK = 128
NCHUNK = COLS_PER_W // CHUNK
GROUPS_PER_CHUNK = CHUNK // LANES
SEARCH_ITERS = 8

_mesh = plsc.VectorSubcoreMesh(core_axis_name="c", subcore_axis_name="s")


@functools.partial(
    pl.kernel,
    out_type=[
        jax.ShapeDtypeStruct((L, B), jnp.float32),
        jax.ShapeDtypeStruct((B,), jnp.int32),
    ],
    mesh=_mesh,
    scratch_types=[
        [pltpu.VMEM((L, CHUNK), jnp.float32)] * NCHUNK,
        pltpu.VMEM((COLS_PER_W,), jnp.int32),
        [pltpu.SemaphoreType.DMA] * NCHUNK,
    ],
    compiler_params=pltpu.CompilerParams(
        needs_layout_passes=False, use_tc_tiling_on_sc=True
    ),
)
def _dense_to_ragged(xt_hbm, vt_hbm, len_hbm, bufs, len_v, sems):
    wid = lax.axis_index("s") * NC + lax.axis_index("c")
    b0 = wid * COLS_PER_W
    iota16 = lax.iota(jnp.int32, 16)

    ins = [
        pltpu.async_copy(
            xt_hbm.at[:, pl.ds(b0 + i * CHUNK, CHUNK)], bufs[i], sems[i]
        )
        for i in range(NCHUNK)
    ]

    outs = []
    for i in range(NCHUNK):
        buf = bufs[i]
        ins[i].wait()

        def grp(g, _):
            cols = g * LANES + iota16
            lo = jnp.zeros((16,), jnp.int32)
            hi = jnp.full((16,), L, jnp.int32)
            for _i in range(SEARCH_ITERS):
                active = lo < hi
                mid = jnp.right_shift(lo + hi, 1)
                midc = jnp.minimum(mid, L - 1)
                v = plsc.load_gather(buf, [midc, cols])
                is_pad = v == PAD
                lo2 = jnp.where(is_pad, lo, mid + 1)
                hi2 = jnp.where(is_pad, mid, hi)
                lo = jnp.where(active, lo2, lo)
                hi = jnp.where(active, hi2, hi)
            len_v[pl.ds(i * CHUNK + g * LANES, 16)] = lo
            return 0

        lax.fori_loop(0, GROUPS_PER_CHUNK, grp, 0)

        @plsc.parallel_loop(0, L, step=1)
        def _ew(l):
            for o in range(0, CHUNK, 16):
                v = buf[l, pl.ds(o, 16)]
                buf[l, pl.ds(o, 16)] = jnp.where(v == PAD, jnp.float32(0.0), v)

        outs.append(
            pltpu.async_copy(
                bufs[i], vt_hbm.at[:, pl.ds(b0 + i * CHUNK, CHUNK)], sems[i]
            )
        )

    for h in outs:
        h.wait()
    pltpu.sync_copy(len_v, len_hbm.at[pl.ds(b0, COLS_PER_W)])


def kernel(inputs):
    values_t, row_lengths = _dense_to_ragged(inputs.T)
    return values_t.T, row_lengths
